# baseline (speedup 1.0000x reference)
import functools

import jax
import jax.numpy as jnp
from jax import lax
from jax.experimental import pallas as pl
from jax.experimental.pallas import tpu as pltpu
from jax.experimental.pallas import tpu_sc as plsc

D_MODEL = 768
N_ROWS = 32768
NC, NS = 2, 16
NW = NC * NS
ROWS_PER_W = N_ROWS // NW
CHUNK = 32
N_CHUNKS = ROWS_PER_W // CHUNK
NBUF = 4


def _gather_kernel(idx_hbm, table_hbm, out_hbm, idx_v, buf0, buf1, buf2, buf3,
                   gsem0, gsem1, gsem2, gsem3, ssem0, ssem1, ssem2, ssem3):
    wid = lax.axis_index("s") * NC + lax.axis_index("c")
    base = wid * ROWS_PER_W
    pltpu.sync_copy(idx_hbm.at[pl.ds(base, ROWS_PER_W)], idx_v)

    bufs = (buf0, buf1, buf2, buf3)
    gsems = (gsem0, gsem1, gsem2, gsem3)
    ssems = (ssem0, ssem1, ssem2, ssem3)
    gathers = [None] * NBUF
    stores = [None] * NBUF

    for c in range(N_CHUNKS):
        b = c % NBUF
        if stores[b] is not None:
            stores[b].wait()
        gathers[b] = pltpu.async_copy(
            table_hbm.at[idx_v.at[pl.ds(c * CHUNK, CHUNK)]], bufs[b], gsems[b])
        if c > 0:
            pb = (c - 1) % NBUF
            gathers[pb].wait()
            stores[pb] = pltpu.async_copy(
                bufs[pb], out_hbm.at[pl.ds(base + (c - 1) * CHUNK, CHUNK)],
                ssems[pb])
    last = N_CHUNKS - 1
    lb = last % NBUF
    gathers[lb].wait()
    stores[lb] = pltpu.async_copy(
        bufs[lb], out_hbm.at[pl.ds(base + last * CHUNK, CHUNK)], ssems[lb])
    for b in range(NBUF):
        if stores[b] is not None:
            stores[b].wait()


@jax.jit
def _lookup(idx_flat, emb_weight):
    mesh = plsc.VectorSubcoreMesh(core_axis_name="c", subcore_axis_name="s")
    k = functools.partial(
        pl.kernel,
        mesh=mesh,
        out_type=jax.ShapeDtypeStruct((N_ROWS, D_MODEL), jnp.float32),
        scratch_types=(
            [pltpu.VMEM((ROWS_PER_W,), jnp.int32)]
            + [pltpu.VMEM((CHUNK, D_MODEL), jnp.float32)] * NBUF
            + [pltpu.SemaphoreType.DMA] * (2 * NBUF)
        ),
    )(_gather_kernel)
    return k(idx_flat, emb_weight)


def kernel(indices, emb_weight):
    batch, seq_len = indices.shape
    idx_flat = indices.reshape(-1).astype(jnp.int32)
    out = _lookup(idx_flat, emb_weight)
    return out.reshape(batch, seq_len, D_MODEL)


# speedup vs baseline: 1.0135x; 1.0135x over previous
import functools

import jax
import jax.numpy as jnp
from jax import lax
from jax.experimental import pallas as pl
from jax.experimental.pallas import tpu as pltpu
from jax.experimental.pallas import tpu_sc as plsc

D_MODEL = 768
N_ROWS = 32768
NC, NS = 2, 16
NW = NC * NS
ROWS_PER_W = N_ROWS // NW
CHUNK = 64
N_CHUNKS = ROWS_PER_W // CHUNK
NBUF = 2


def _gather_kernel(idx_hbm, table_hbm, out_hbm, idx_v, buf0, buf1,
                   gsem0, gsem1, ssem0, ssem1):
    wid = lax.axis_index("s") * NC + lax.axis_index("c")
    base = wid * ROWS_PER_W
    pltpu.sync_copy(idx_hbm.at[pl.ds(base, ROWS_PER_W)], idx_v)

    bufs = (buf0, buf1)
    gsems = (gsem0, gsem1)
    ssems = (ssem0, ssem1)
    gathers = [None] * NBUF
    stores = [None] * NBUF

    for c in range(N_CHUNKS):
        b = c % NBUF
        if stores[b] is not None:
            stores[b].wait()
        gathers[b] = pltpu.async_copy(
            table_hbm.at[idx_v.at[pl.ds(c * CHUNK, CHUNK)]], bufs[b], gsems[b])
        if c > 0:
            pb = (c - 1) % NBUF
            gathers[pb].wait()
            stores[pb] = pltpu.async_copy(
                bufs[pb], out_hbm.at[pl.ds(base + (c - 1) * CHUNK, CHUNK)],
                ssems[pb])
    last = N_CHUNKS - 1
    lb = last % NBUF
    gathers[lb].wait()
    stores[lb] = pltpu.async_copy(
        bufs[lb], out_hbm.at[pl.ds(base + last * CHUNK, CHUNK)], ssems[lb])
    for b in range(NBUF):
        if stores[b] is not None:
            stores[b].wait()


@jax.jit
def _lookup(idx_flat, emb_weight):
    mesh = plsc.VectorSubcoreMesh(core_axis_name="c", subcore_axis_name="s")
    k = functools.partial(
        pl.kernel,
        mesh=mesh,
        out_type=jax.ShapeDtypeStruct((N_ROWS, D_MODEL), jnp.float32),
        scratch_types=(
            [pltpu.VMEM((ROWS_PER_W,), jnp.int32)]
            + [pltpu.VMEM((CHUNK, D_MODEL), jnp.float32)] * NBUF
            + [pltpu.SemaphoreType.DMA] * (2 * NBUF)
        ),
    )(_gather_kernel)
    return k(idx_flat, emb_weight)


def kernel(indices, emb_weight):
    batch, seq_len = indices.shape
    idx_flat = indices.reshape(-1).astype(jnp.int32)
    out = _lookup(idx_flat, emb_weight)
    return out.reshape(batch, seq_len, D_MODEL)
